# baseline (speedup 1.0000x reference)
import functools

import jax
import jax.numpy as jnp
from jax import lax
from jax.experimental import pallas as pl
from jax.experimental.pallas import tpu as pltpu
from jax.experimental.pallas import tpu_sc as plsc

F = 26
VOCAB = 100000
EMB = 32
B = 1024
T = 200

BT = B * T
NC = 2
NS = 16
NW = NC * NS
ROWS_W = BT // NW
Q = 64
NCHUNK = ROWS_W // Q
NPAIR = NCHUNK // 2
SLAB = F * Q

_mesh = plsc.VectorSubcoreMesh(
    core_axis_name="c", subcore_axis_name="s", num_cores=NC, num_subcores=NS
)


@functools.partial(
    pl.kernel,
    mesh=_mesh,
    out_type=jax.ShapeDtypeStruct((BT, F * EMB), jnp.float32),
    compiler_params=pltpu.CompilerParams(use_tc_tiling_on_sc=False),
    scratch_types=[
        pltpu.VMEM((2, F, Q), jnp.int32),
        pltpu.VMEM((2 * SLAB, EMB), jnp.float32),
        pltpu.SemaphoreType.DMA,
        pltpu.SemaphoreType.DMA,
        pltpu.SemaphoreType.DMA,
        pltpu.SemaphoreType.DMA,
    ],
)
def _gather_kernel(
    table_hbm, gidx_hbm, out_hbm, idx_v, stages, semg, semi, semw0, semw1
):
    wid = lax.axis_index("s") * NC + lax.axis_index("c")
    semw = (semw0, semw1)

    def one_chunk(g, parity, first_pair):
        buf = parity * SLAB
        bt0 = wid * ROWS_W + g * Q

        @pl.when(jnp.logical_or(g > 0, False))
        def _():
            pltpu.make_async_copy(
                gidx_hbm.at[wid, 0], idx_v.at[parity], semi
            ).wait()

        nxt = jnp.minimum(g + 1, NCHUNK - 1)
        pltpu.async_copy(gidx_hbm.at[wid, nxt], idx_v.at[1 - parity], semi)

        @pl.when(jnp.logical_not(first_pair))
        def _():
            pltpu.make_async_copy(
                stages.at[pl.ds(0, SLAB)], out_hbm.at[pl.ds(0, Q)], semw[parity]
            ).wait()

        def fire_gather(f, c):
            pltpu.async_copy(
                table_hbm.at[idx_v.at[parity, f]],
                stages.at[pl.ds(buf + f * Q, Q)],
                semg,
            )
            return c

        lax.fori_loop(0, F, fire_gather, 0)
        pltpu.make_async_copy(
            table_hbm.at[pl.ds(0, SLAB)], stages.at[pl.ds(0, SLAB)], semg
        ).wait()

        def fire_write(f, c):
            pltpu.async_copy(
                stages.at[pl.ds(buf + f * Q, Q)],
                out_hbm.at[pl.ds(bt0, Q), pl.ds(f * EMB, EMB)],
                semw[parity],
            )
            return c

        lax.fori_loop(0, F, fire_write, 0)

    pltpu.sync_copy(gidx_hbm.at[wid, 0], idx_v.at[0])

    def pair_body(gp, carry):
        one_chunk(2 * gp, 0, gp == 0)
        one_chunk(2 * gp + 1, 1, gp == 0)
        return carry

    lax.fori_loop(0, NPAIR, pair_body, 0)
    pltpu.make_async_copy(gidx_hbm.at[wid, 0], idx_v.at[0], semi).wait()
    pltpu.make_async_copy(
        stages.at[pl.ds(0, SLAB)], out_hbm.at[pl.ds(0, Q)], semw0
    ).wait()
    pltpu.make_async_copy(
        stages.at[pl.ds(0, SLAB)], out_hbm.at[pl.ds(0, Q)], semw1
    ).wait()


def kernel(tables, indices, seq_lens):
    table_flat = tables.reshape(F * VOCAB, EMB)
    offs = jnp.arange(F, dtype=jnp.int32) * VOCAB
    x = indices + offs[None, None, :]
    gidx4 = (
        x.reshape(NW, NCHUNK, Q, F).transpose(0, 1, 3, 2)
    )
    out2 = _gather_kernel(table_flat, gidx4)
    return out2.reshape(B, T, F * EMB)


# speedup vs baseline: 1.0284x; 1.0284x over previous
import functools

import jax
import jax.numpy as jnp
from jax import lax
from jax.experimental import pallas as pl
from jax.experimental.pallas import tpu as pltpu
from jax.experimental.pallas import tpu_sc as plsc

F = 26
VOCAB = 100000
EMB = 32
B = 1024
T = 200

BT = B * T
NC = 2
NS = 16
NW = NC * NS
ROWS_W = BT // NW
Q = 64
NCHUNK = ROWS_W // Q
NPAIR = NCHUNK // 2
SLAB = F * Q

_mesh = plsc.VectorSubcoreMesh(
    core_axis_name="c", subcore_axis_name="s", num_cores=NC, num_subcores=NS
)


@functools.partial(
    pl.kernel,
    mesh=_mesh,
    out_type=jax.ShapeDtypeStruct((BT, F * EMB), jnp.float32),
    compiler_params=pltpu.CompilerParams(use_tc_tiling_on_sc=False),
    scratch_types=[
        pltpu.VMEM((2, F, Q), jnp.int32),
        pltpu.VMEM((2 * SLAB, EMB), jnp.float32),
        pltpu.SemaphoreType.DMA,
        pltpu.SemaphoreType.DMA,
        pltpu.SemaphoreType.DMA,
    ],
)
def _gather_kernel(table_hbm, gidx_hbm, out_hbm, idx_v, stages, semg, semw0, semw1):
    wid = lax.axis_index("s") * NC + lax.axis_index("c")
    semw = (semw0, semw1)

    def one_chunk(g, parity, first):
        buf = parity * SLAB
        bt0 = wid * ROWS_W + g * Q
        pltpu.sync_copy(gidx_hbm.at[:, pl.ds(bt0, Q)], idx_v.at[parity])

        @pl.when(jnp.logical_not(first))
        def _():
            pltpu.make_async_copy(
                stages.at[pl.ds(0, SLAB)], out_hbm.at[pl.ds(0, Q)], semw[parity]
            ).wait()

        def fire_gather(f, c):
            pltpu.async_copy(
                table_hbm.at[idx_v.at[parity, f]],
                stages.at[pl.ds(buf + f * Q, Q)],
                semg,
            )
            return c

        lax.fori_loop(0, F, fire_gather, 0)
        pltpu.make_async_copy(
            table_hbm.at[pl.ds(0, SLAB)], stages.at[pl.ds(0, SLAB)], semg
        ).wait()

        def fire_write(f, c):
            pltpu.async_copy(
                stages.at[pl.ds(buf + f * Q, Q)],
                out_hbm.at[pl.ds(bt0, Q), pl.ds(f * EMB, EMB)],
                semw[parity],
            )
            return c

        lax.fori_loop(0, F, fire_write, 0)

    def pair_body(gp, carry):
        one_chunk(2 * gp, 0, gp == 0)
        one_chunk(2 * gp + 1, 1, gp == 0)
        return carry

    lax.fori_loop(0, NPAIR, pair_body, 0)
    pltpu.make_async_copy(
        stages.at[pl.ds(0, SLAB)], out_hbm.at[pl.ds(0, Q)], semw0
    ).wait()
    pltpu.make_async_copy(
        stages.at[pl.ds(0, SLAB)], out_hbm.at[pl.ds(0, Q)], semw1
    ).wait()


def kernel(tables, indices, seq_lens):
    table_flat = tables.reshape(F * VOCAB, EMB)
    offs = jnp.arange(F, dtype=jnp.int32) * VOCAB
    gidx_t = (indices + offs[None, None, :]).transpose(2, 0, 1).reshape(F, BT)
    out2 = _gather_kernel(table_flat, gidx_t)
    return out2.reshape(B, T, F * EMB)
